# baseline (speedup 1.0000x reference)
import functools

import jax
import jax.numpy as jnp
from jax import lax
from jax.experimental import pallas as pl
from jax.experimental.pallas import tpu as pltpu
from jax.experimental.pallas import tpu_sc as plsc

N = 10000
E = 320000
IN_DIM = 32
HID = 128
NLAYERS = 4
NCLS = 6

NC = 2
NS = 16
NW = NC * NS
CHUNK = 128
NCHUNK = 80
EPW = NCHUNK * CHUNK
EP = NW * EPW
HID2 = HID // 2
NCHUNK2 = EP // (NS * CHUNK)
SUPER = 2
NSUP = NCHUNK2 // SUPER
NBUF = 2
ROWS_PER_SUB = 640
NP = ROWS_PER_SUB * NS

_mesh = plsc.VectorSubcoreMesh(
    core_axis_name="c", subcore_axis_name="s", num_cores=NC, num_subcores=NS)


@functools.partial(
    pl.kernel,
    out_type=jax.ShapeDtypeStruct((NC * NP,), jnp.float32),
    mesh=_mesh,
    scratch_types=[
        pltpu.VMEM((NCHUNK, CHUNK), jnp.int32),
        pltpu.VMEM((CHUNK,), jnp.float32),
        pltpu.VMEM((ROWS_PER_SUB,), jnp.float32),
        pltpu.VMEM_SHARED((NP,), jnp.float32),
    ],
)
def _deg_kernel(dst_hbm, out_hbm, dst_v, ones_v, buf_v, deg_sh):
    c = lax.axis_index("c")
    s = lax.axis_index("s")
    wid = s * NC + c
    for i in range(ROWS_PER_SUB // 16):
        buf_v[pl.ds(i * 16, 16)] = jnp.zeros((16,), jnp.float32)
    pltpu.sync_copy(buf_v, deg_sh.at[pl.ds(s * ROWS_PER_SUB, ROWS_PER_SUB)])
    pltpu.sync_copy(dst_hbm.at[wid], dst_v)
    for i in range(CHUNK // 16):
        ones_v[pl.ds(i * 16, 16)] = jnp.ones((16,), jnp.float32)
    plsc.subcore_barrier()

    def body(j, carry):
        pltpu.sync_copy(ones_v, deg_sh.at[dst_v.at[j]], add=True)
        return carry

    lax.fori_loop(0, NCHUNK, body, 0)
    plsc.subcore_barrier()
    pltpu.sync_copy(deg_sh.at[pl.ds(s * ROWS_PER_SUB, ROWS_PER_SUB)], buf_v)
    pltpu.sync_copy(buf_v,
                    out_hbm.at[pl.ds(c * NP + s * ROWS_PER_SUB, ROWS_PER_SUB)])


@functools.partial(
    pl.kernel,
    out_type=jax.ShapeDtypeStruct((NC, NP, HID2), jnp.float32),
    mesh=_mesh,
    compiler_params=pltpu.CompilerParams(use_tc_tiling_on_sc=False),
    scratch_types=[
        pltpu.VMEM((NSUP, SUPER * CHUNK), jnp.int32),
        pltpu.VMEM((NSUP, SUPER * CHUNK), jnp.int32),
        [pltpu.VMEM((SUPER * CHUNK, HID2), jnp.float32)] * NBUF,
        pltpu.VMEM_SHARED((NP, HID2), jnp.float32),
        [pltpu.SemaphoreType.DMA] * NBUF,
        [pltpu.SemaphoreType.DMA] * NBUF,
    ],
)
def _agg_kernel(xn_hbm, src_hbm, dst_hbm, zeros_hbm, out_hbm,
                src_v, dst_v, rows, acc_sh, gsem, ssem):
    B = NBUF
    NG = NSUP // B
    c = lax.axis_index("c")
    s = lax.axis_index("s")
    pltpu.sync_copy(zeros_hbm, acc_sh.at[pl.ds(s * ROWS_PER_SUB, ROWS_PER_SUB)])
    pltpu.sync_copy(src_hbm.at[s], src_v)
    pltpu.sync_copy(dst_hbm.at[s], dst_v)
    plsc.subcore_barrier()

    xnc = xn_hbm.at[c]

    for b in range(B):
        pltpu.async_copy(xnc.at[src_v.at[b]], rows[b], gsem[b])

    def outer(g, carry):
        for b in range(B):
            j = g * B + b
            pltpu.make_async_copy(xnc.at[src_v.at[j]], rows[b], gsem[b]).wait()
            pltpu.async_copy(rows[b], acc_sh.at[dst_v.at[j]], ssem[b], add=True)
        for b in range(B):
            j = g * B + b
            jn = jnp.minimum(j + B, NSUP - 1)

            @pl.when(g < NG - 1)
            def _(b=b, j=j, jn=jn):
                pltpu.make_async_copy(rows[b], acc_sh.at[dst_v.at[j]], ssem[b]).wait()
                pltpu.async_copy(xnc.at[src_v.at[jn]], rows[b], gsem[b])
        return carry

    lax.fori_loop(0, NG, outer, 0)
    for b in range(B):
        pltpu.make_async_copy(rows[b], acc_sh.at[dst_v.at[NSUP - 1]], ssem[b]).wait()
    plsc.subcore_barrier()
    pltpu.sync_copy(acc_sh.at[pl.ds(s * ROWS_PER_SUB, ROWS_PER_SUB)],
                    out_hbm.at[c, pl.ds(s * ROWS_PER_SUB, ROWS_PER_SUB)])


def _prep_body(h_ref, degp_ref, emb_ref, x_ref, xn_ref, norm_ref):
    d = degp_ref[...]
    deg = d[:, 0:1] + d[:, 1:2]
    norm = lax.rsqrt(jnp.maximum(deg, 1.0))
    hv = h_ref[...]
    oh = (hv == lax.broadcasted_iota(jnp.int32, (N, IN_DIM), 1)).astype(jnp.float32)
    x = jnp.dot(oh, emb_ref[...], preferred_element_type=jnp.float32, precision=lax.Precision.HIGHEST)
    x_ref[...] = x
    xnv = x * norm
    xn_ref[0, :, :] = xnv[:, :HID2]
    xn_ref[1, :, :] = xnv[:, HID2:]
    norm_ref[...] = norm


_prep_call = pl.pallas_call(
    _prep_body,
    out_shape=[
        jax.ShapeDtypeStruct((N, HID), jnp.float32),
        jax.ShapeDtypeStruct((NC, N, HID2), jnp.float32),
        jax.ShapeDtypeStruct((N, 1), jnp.float32),
    ],
)


def _dot_bf16(a, b):
    return jnp.dot(a.astype(jnp.bfloat16), b.astype(jnp.bfloat16),
                   preferred_element_type=jnp.float32)


def _layer_body(x_ref, ap_ref, norm_ref, W_ref, b_ref, g_ref, be_ref,
                xo_ref, xn_ref):
    norm = norm_ref[...]
    agg = jnp.concatenate([ap_ref[0, :N, :], ap_ref[1, :N, :]], axis=1) * norm
    y = _dot_bf16(agg, W_ref[...]) + b_ref[...]
    mean = jnp.mean(y, axis=0, keepdims=True)
    yc = y - mean
    var = jnp.mean(yc * yc, axis=0, keepdims=True)
    y = yc * lax.rsqrt(var + 1e-5) * g_ref[...] + be_ref[...]
    y = jnp.maximum(y, 0.0)
    xo = x_ref[...] + y
    xo_ref[...] = xo
    xnv = xo * norm
    xn_ref[0, :, :] = xnv[:, :HID2]
    xn_ref[1, :, :] = xnv[:, HID2:]


_layer_call = pl.pallas_call(
    _layer_body,
    out_shape=[
        jax.ShapeDtypeStruct((N, HID), jnp.float32),
        jax.ShapeDtypeStruct((NC, N, HID2), jnp.float32),
    ],
)


def _mlp_body(x_ref, W0_ref, b0_ref, W1_ref, b1_ref, W2_ref, b2_ref, o_ref):
    z = _dot_bf16(x_ref[...], W0_ref[...])
    z = jnp.maximum(z + b0_ref[...], 0.0)
    z = _dot_bf16(z, W1_ref[...])
    z = jnp.maximum(z + b1_ref[...], 0.0)
    o_ref[...] = _dot_bf16(z, W2_ref[...]) + b2_ref[...]


_mlp_call = pl.pallas_call(
    _mlp_body,
    out_shape=jax.ShapeDtypeStruct((N, NCLS), jnp.float32),
)


def kernel(h, edge_index, e, emb, W_layers, b_layers, bn_scale, bn_bias,
           mlp_W0, mlp_b0, mlp_W1, mlp_b1, mlp_W2, mlp_b2):
    src = edge_index[0].astype(jnp.int32)
    dst = edge_index[1].astype(jnp.int32)
    pad = EP - E
    src_flat = jnp.concatenate([src, jnp.zeros((pad,), jnp.int32)])
    dummy = N + (jnp.arange(pad, dtype=jnp.int32) % (NP - N))
    dst_flat = jnp.concatenate([dst, dummy])
    src_p = src_flat.reshape(NW, NCHUNK, CHUNK)
    dst_p = dst_flat.reshape(NW, NCHUNK, CHUNK)
    src_p2 = src_flat.reshape(NS, NSUP, SUPER * CHUNK)
    dst_p2 = dst_flat.reshape(NS, NSUP, SUPER * CHUNK)
    zeros2 = jnp.zeros((ROWS_PER_SUB, HID2), jnp.float32)

    degp = _deg_kernel(dst_p).reshape(NC, NP)
    degp_t = jnp.transpose(degp[:, :N])
    hv = h.astype(jnp.int32).reshape(N, 1)
    x, xn, norm = _prep_call(hv, degp_t, emb)

    for i in range(NLAYERS):
        aggp = _agg_kernel(xn, src_p2, dst_p2, zeros2)
        x, xn = _layer_call(x, aggp, norm, W_layers[i],
                            b_layers[i].reshape(1, HID),
                            bn_scale[i].reshape(1, HID),
                            bn_bias[i].reshape(1, HID))

    out = _mlp_call(x, mlp_W0, mlp_b0.reshape(1, -1),
                    mlp_W1, mlp_b1.reshape(1, -1),
                    mlp_W2, mlp_b2.reshape(1, -1))
    return out


# speedup vs baseline: 1.0720x; 1.0720x over previous
import functools

import jax
import jax.numpy as jnp
from jax import lax
from jax.experimental import pallas as pl
from jax.experimental.pallas import tpu as pltpu
from jax.experimental.pallas import tpu_sc as plsc

N = 10000
E = 320000
IN_DIM = 32
HID = 128
NLAYERS = 4
NCLS = 6

NC = 2
NS = 16
NW = NC * NS
CHUNK = 128
NCHUNK = 80
EPW = NCHUNK * CHUNK
EP = NW * EPW
HID2 = HID // 2
SCH = 512
NSUP = EP // (NS * SCH)
W = 4
NWIN = NSUP // W
NT = NWIN // 2
ROWS_PER_SUB = 640
NP = ROWS_PER_SUB * NS

_mesh = plsc.VectorSubcoreMesh(
    core_axis_name="c", subcore_axis_name="s", num_cores=NC, num_subcores=NS)


@functools.partial(
    pl.kernel,
    out_type=jax.ShapeDtypeStruct((NC * NP,), jnp.float32),
    mesh=_mesh,
    scratch_types=[
        pltpu.VMEM((NCHUNK, CHUNK), jnp.int32),
        pltpu.VMEM((CHUNK,), jnp.float32),
        pltpu.VMEM((ROWS_PER_SUB,), jnp.float32),
        pltpu.VMEM_SHARED((NP,), jnp.float32),
    ],
)
def _deg_kernel(dst_hbm, out_hbm, dst_v, ones_v, buf_v, deg_sh):
    c = lax.axis_index("c")
    s = lax.axis_index("s")
    wid = s * NC + c
    for i in range(ROWS_PER_SUB // 16):
        buf_v[pl.ds(i * 16, 16)] = jnp.zeros((16,), jnp.float32)
    pltpu.sync_copy(buf_v, deg_sh.at[pl.ds(s * ROWS_PER_SUB, ROWS_PER_SUB)])
    pltpu.sync_copy(dst_hbm.at[wid], dst_v)
    for i in range(CHUNK // 16):
        ones_v[pl.ds(i * 16, 16)] = jnp.ones((16,), jnp.float32)
    plsc.subcore_barrier()

    def body(j, carry):
        pltpu.sync_copy(ones_v, deg_sh.at[dst_v.at[j]], add=True)
        return carry

    lax.fori_loop(0, NCHUNK, body, 0)
    plsc.subcore_barrier()
    pltpu.sync_copy(deg_sh.at[pl.ds(s * ROWS_PER_SUB, ROWS_PER_SUB)], buf_v)
    pltpu.sync_copy(buf_v,
                    out_hbm.at[pl.ds(c * NP + s * ROWS_PER_SUB, ROWS_PER_SUB)])


@functools.partial(
    pl.kernel,
    out_type=jax.ShapeDtypeStruct((NC, NP, HID2), jnp.float32),
    mesh=_mesh,
    compiler_params=pltpu.CompilerParams(use_tc_tiling_on_sc=False),
    scratch_types=[
        [pltpu.VMEM((W, SCH), jnp.int32)] * 2,
        [pltpu.VMEM((W, SCH), jnp.int32)] * 2,
        [pltpu.VMEM((SCH, HID2), jnp.float32)] * 2,
        pltpu.VMEM_SHARED((NP, HID2), jnp.float32),
        [pltpu.SemaphoreType.DMA] * 2,
        [pltpu.SemaphoreType.DMA] * 2,
        [pltpu.SemaphoreType.DMA] * 2,
    ],
)
def _agg_kernel(xn_hbm, src_hbm, dst_hbm, zeros_hbm, out_hbm,
                srcw, dstw, rows, acc_sh, gsem, isrc, idst):
    c = lax.axis_index("c")
    s = lax.axis_index("s")
    pltpu.sync_copy(zeros_hbm, acc_sh.at[pl.ds(s * ROWS_PER_SUB, ROWS_PER_SUB)])
    pltpu.sync_copy(src_hbm.at[s, 0], srcw[0])
    pltpu.sync_copy(dst_hbm.at[s, 0], dstw[0])
    pltpu.async_copy(src_hbm.at[s, 1], srcw[1], isrc[1])
    pltpu.async_copy(dst_hbm.at[s, 1], dstw[1], idst[1])
    plsc.subcore_barrier()

    xnc = xn_hbm.at[c]
    pltpu.async_copy(xnc.at[srcw[0].at[0]], rows[0], gsem[0])
    pltpu.async_copy(xnc.at[srcw[0].at[1]], rows[1], gsem[1])

    def body(t, carry):
        for p in range(2):
            if p == 0:
                @pl.when(t > 0)
                def _():
                    pltpu.make_async_copy(dst_hbm.at[s, 0], dstw[0], idst[0]).wait()
            else:
                pltpu.make_async_copy(dst_hbm.at[s, 0], dstw[1], idst[1]).wait()
            for k in range(W):
                b = k % 2
                pltpu.make_async_copy(xnc.at[srcw[p].at[k]], rows[b], gsem[b]).wait()
                pltpu.sync_copy(rows[b], acc_sh.at[dstw[p].at[k]], add=True)
                if k < 2:
                    pltpu.async_copy(xnc.at[srcw[p].at[k + 2]], rows[b], gsem[b])
                elif p == 0:
                    if k == 2:
                        pltpu.make_async_copy(src_hbm.at[s, 1], srcw[1], isrc[1]).wait()
                    pltpu.async_copy(xnc.at[srcw[1].at[k - 2]], rows[b], gsem[b])
                else:
                    @pl.when(t < NT - 1)
                    def _(b=b, k=k):
                        if k == 2:
                            pltpu.make_async_copy(
                                src_hbm.at[s, 0], srcw[0], isrc[0]).wait()
                        pltpu.async_copy(xnc.at[srcw[0].at[k - 2]], rows[b], gsem[b])
            @pl.when(t < NT - 1)
            def _(p=p, t=t):
                wn = 2 * t + 2 + p
                pltpu.async_copy(src_hbm.at[s, wn], srcw[p], isrc[p])
                pltpu.async_copy(dst_hbm.at[s, wn], dstw[p], idst[p])
        return carry

    lax.fori_loop(0, NT, body, 0)
    plsc.subcore_barrier()
    pltpu.sync_copy(acc_sh.at[pl.ds(s * ROWS_PER_SUB, ROWS_PER_SUB)],
                    out_hbm.at[c, pl.ds(s * ROWS_PER_SUB, ROWS_PER_SUB)])


def _prep_body(h_ref, degp_ref, emb_ref, x_ref, xn_ref, norm_ref):
    d = degp_ref[...]
    deg = d[:, 0:1] + d[:, 1:2]
    norm = lax.rsqrt(jnp.maximum(deg, 1.0))
    hv = h_ref[...]
    oh = (hv == lax.broadcasted_iota(jnp.int32, (N, IN_DIM), 1)).astype(jnp.float32)
    x = jnp.dot(oh, emb_ref[...], preferred_element_type=jnp.float32, precision=lax.Precision.HIGHEST)
    x_ref[...] = x
    xnv = x * norm
    xn_ref[0, :, :] = xnv[:, :HID2]
    xn_ref[1, :, :] = xnv[:, HID2:]
    norm_ref[...] = norm


_prep_call = pl.pallas_call(
    _prep_body,
    out_shape=[
        jax.ShapeDtypeStruct((N, HID), jnp.float32),
        jax.ShapeDtypeStruct((NC, N, HID2), jnp.float32),
        jax.ShapeDtypeStruct((N, 1), jnp.float32),
    ],
)


def _dot_bf16(a, b):
    return jnp.dot(a.astype(jnp.bfloat16), b.astype(jnp.bfloat16),
                   preferred_element_type=jnp.float32)


def _layer_body(x_ref, ap_ref, norm_ref, W_ref, b_ref, g_ref, be_ref,
                xo_ref, xn_ref):
    norm = norm_ref[...]
    agg = jnp.concatenate([ap_ref[0, :N, :], ap_ref[1, :N, :]], axis=1) * norm
    y = _dot_bf16(agg, W_ref[...]) + b_ref[...]
    mean = jnp.mean(y, axis=0, keepdims=True)
    yc = y - mean
    var = jnp.mean(yc * yc, axis=0, keepdims=True)
    y = yc * lax.rsqrt(var + 1e-5) * g_ref[...] + be_ref[...]
    y = jnp.maximum(y, 0.0)
    xo = x_ref[...] + y
    xo_ref[...] = xo
    xnv = xo * norm
    xn_ref[0, :, :] = xnv[:, :HID2]
    xn_ref[1, :, :] = xnv[:, HID2:]


_layer_call = pl.pallas_call(
    _layer_body,
    out_shape=[
        jax.ShapeDtypeStruct((N, HID), jnp.float32),
        jax.ShapeDtypeStruct((NC, N, HID2), jnp.float32),
    ],
)


def _mlp_body(x_ref, W0_ref, b0_ref, W1_ref, b1_ref, W2_ref, b2_ref, o_ref):
    z = _dot_bf16(x_ref[...], W0_ref[...])
    z = jnp.maximum(z + b0_ref[...], 0.0)
    z = _dot_bf16(z, W1_ref[...])
    z = jnp.maximum(z + b1_ref[...], 0.0)
    o_ref[...] = _dot_bf16(z, W2_ref[...]) + b2_ref[...]


_mlp_call = pl.pallas_call(
    _mlp_body,
    out_shape=jax.ShapeDtypeStruct((N, NCLS), jnp.float32),
)


def kernel(h, edge_index, e, emb, W_layers, b_layers, bn_scale, bn_bias,
           mlp_W0, mlp_b0, mlp_W1, mlp_b1, mlp_W2, mlp_b2):
    src = edge_index[0].astype(jnp.int32)
    dst = edge_index[1].astype(jnp.int32)
    pad = EP - E
    src_flat = jnp.concatenate([src, jnp.zeros((pad,), jnp.int32)])
    dummy = N + (jnp.arange(pad, dtype=jnp.int32) % (NP - N))
    dst_flat = jnp.concatenate([dst, dummy])
    src_p = src_flat.reshape(NW, NCHUNK, CHUNK)
    dst_p = dst_flat.reshape(NW, NCHUNK, CHUNK)
    src_p2 = src_flat.reshape(NS, NWIN, W, SCH)
    dst_p2 = dst_flat.reshape(NS, NWIN, W, SCH)
    zeros2 = jnp.zeros((ROWS_PER_SUB, HID2), jnp.float32)

    degp = _deg_kernel(dst_p).reshape(NC, NP)
    degp_t = jnp.transpose(degp[:, :N])
    hv = h.astype(jnp.int32).reshape(N, 1)
    x, xn, norm = _prep_call(hv, degp_t, emb)

    for i in range(NLAYERS):
        aggp = _agg_kernel(xn, src_p2, dst_p2, zeros2)
        x, xn = _layer_call(x, aggp, norm, W_layers[i],
                            b_layers[i].reshape(1, HID),
                            bn_scale[i].reshape(1, HID),
                            bn_bias[i].reshape(1, HID))

    out = _mlp_call(x, mlp_W0, mlp_b0.reshape(1, -1),
                    mlp_W1, mlp_b1.reshape(1, -1),
                    mlp_W2, mlp_b2.reshape(1, -1))
    return out


# speedup vs baseline: 1.2716x; 1.1862x over previous
import functools

import jax
import jax.numpy as jnp
from jax import lax
from jax.experimental import pallas as pl
from jax.experimental.pallas import tpu as pltpu
from jax.experimental.pallas import tpu_sc as plsc

N = 10000
E = 320000
IN_DIM = 32
HID = 128
NLAYERS = 4
NCLS = 6

NC = 2
NS = 16
NW = NC * NS
CHUNK = 128
NCHUNK = 80
EPW = NCHUNK * CHUNK
EP = NW * EPW
HID2 = HID // 2
SCH = 512
NSUP = EP // (NS * SCH)
W = 4
NWIN = NSUP // W
NT = NWIN // 2
ROWS_PER_SUB = 640
NP = ROWS_PER_SUB * NS

_mesh = plsc.VectorSubcoreMesh(
    core_axis_name="c", subcore_axis_name="s", num_cores=NC, num_subcores=NS)


@functools.partial(
    pl.kernel,
    out_type=jax.ShapeDtypeStruct((NC * NP,), jnp.float32),
    mesh=_mesh,
    scratch_types=[
        pltpu.VMEM((NCHUNK, CHUNK), jnp.int32),
        pltpu.VMEM((CHUNK,), jnp.float32),
        pltpu.VMEM((ROWS_PER_SUB,), jnp.float32),
        pltpu.VMEM_SHARED((NP,), jnp.float32),
    ],
)
def _deg_kernel(dst_hbm, out_hbm, dst_v, ones_v, buf_v, deg_sh):
    c = lax.axis_index("c")
    s = lax.axis_index("s")
    wid = s * NC + c
    for i in range(ROWS_PER_SUB // 16):
        buf_v[pl.ds(i * 16, 16)] = jnp.zeros((16,), jnp.float32)
    pltpu.sync_copy(buf_v, deg_sh.at[pl.ds(s * ROWS_PER_SUB, ROWS_PER_SUB)])
    pltpu.sync_copy(dst_hbm.at[wid], dst_v)
    for i in range(CHUNK // 16):
        ones_v[pl.ds(i * 16, 16)] = jnp.ones((16,), jnp.float32)
    plsc.subcore_barrier()

    def body(j, carry):
        pltpu.sync_copy(ones_v, deg_sh.at[dst_v.at[j]], add=True)
        return carry

    lax.fori_loop(0, NCHUNK, body, 0)
    plsc.subcore_barrier()
    pltpu.sync_copy(deg_sh.at[pl.ds(s * ROWS_PER_SUB, ROWS_PER_SUB)], buf_v)
    pltpu.sync_copy(buf_v,
                    out_hbm.at[pl.ds(c * NP + s * ROWS_PER_SUB, ROWS_PER_SUB)])


def _make_agg(ncols):
  @functools.partial(
      pl.kernel,
      out_type=jax.ShapeDtypeStruct((NC, NP, ncols), jnp.float32),
      mesh=_mesh,
      compiler_params=pltpu.CompilerParams(use_tc_tiling_on_sc=False),
      scratch_types=[
          [pltpu.VMEM((W, SCH), jnp.int32)] * 2,
          [pltpu.VMEM((W, SCH), jnp.int32)] * 2,
          [pltpu.VMEM((SCH, ncols), jnp.float32)] * 2,
          pltpu.VMEM_SHARED((NP, ncols), jnp.float32),
          [pltpu.SemaphoreType.DMA] * 2,
          [pltpu.SemaphoreType.DMA] * 2,
          [pltpu.SemaphoreType.DMA] * 2,
      ],
  )
  def _agg_kernel(xn_hbm, src_hbm, dst_hbm, zeros_hbm, out_hbm,
                  srcw, dstw, rows, acc_sh, gsem, isrc, idst):
    c = lax.axis_index("c")
    s = lax.axis_index("s")
    pltpu.sync_copy(zeros_hbm, acc_sh.at[pl.ds(s * ROWS_PER_SUB, ROWS_PER_SUB)])
    pltpu.sync_copy(src_hbm.at[s, 0], srcw[0])
    pltpu.sync_copy(dst_hbm.at[s, 0], dstw[0])
    pltpu.async_copy(src_hbm.at[s, 1], srcw[1], isrc[1])
    pltpu.async_copy(dst_hbm.at[s, 1], dstw[1], idst[1])
    plsc.subcore_barrier()

    xnc = xn_hbm.at[c]
    pltpu.async_copy(xnc.at[srcw[0].at[0]], rows[0], gsem[0])
    pltpu.async_copy(xnc.at[srcw[0].at[1]], rows[1], gsem[1])

    def body(t, carry):
        for p in range(2):
            if p == 0:
                @pl.when(t > 0)
                def _():
                    pltpu.make_async_copy(dst_hbm.at[s, 0], dstw[0], idst[0]).wait()
            else:
                pltpu.make_async_copy(dst_hbm.at[s, 0], dstw[1], idst[1]).wait()
            for k in range(W):
                b = k % 2
                pltpu.make_async_copy(xnc.at[srcw[p].at[k]], rows[b], gsem[b]).wait()
                pltpu.sync_copy(rows[b], acc_sh.at[dstw[p].at[k]], add=True)
                if k < 2:
                    pltpu.async_copy(xnc.at[srcw[p].at[k + 2]], rows[b], gsem[b])
                elif p == 0:
                    if k == 2:
                        pltpu.make_async_copy(src_hbm.at[s, 1], srcw[1], isrc[1]).wait()
                    pltpu.async_copy(xnc.at[srcw[1].at[k - 2]], rows[b], gsem[b])
                else:
                    @pl.when(t < NT - 1)
                    def _(b=b, k=k):
                        if k == 2:
                            pltpu.make_async_copy(
                                src_hbm.at[s, 0], srcw[0], isrc[0]).wait()
                        pltpu.async_copy(xnc.at[srcw[0].at[k - 2]], rows[b], gsem[b])
            @pl.when(t < NT - 1)
            def _(p=p, t=t):
                wn = 2 * t + 2 + p
                pltpu.async_copy(src_hbm.at[s, wn], srcw[p], isrc[p])
                pltpu.async_copy(dst_hbm.at[s, wn], dstw[p], idst[p])
        return carry

    lax.fori_loop(0, NT, body, 0)
    plsc.subcore_barrier()
    pltpu.sync_copy(acc_sh.at[pl.ds(s * ROWS_PER_SUB, ROWS_PER_SUB)],
                    out_hbm.at[c, pl.ds(s * ROWS_PER_SUB, ROWS_PER_SUB)])

  return _agg_kernel


IN2 = IN_DIM // NC
_agg64 = _make_agg(HID2)
_agg16 = _make_agg(IN2)


def _prep_body(h_ref, degp_ref, emb_ref, x_ref, xn_ref, norm_ref):
    d = degp_ref[...]
    deg = d[:, 0:1] + d[:, 1:2]
    norm = lax.rsqrt(jnp.maximum(deg, 1.0))
    hv = h_ref[...]
    oh = (hv == lax.broadcasted_iota(jnp.int32, (N, IN_DIM), 1)).astype(jnp.float32)
    x = jnp.dot(oh, emb_ref[...], preferred_element_type=jnp.float32, precision=lax.Precision.HIGHEST)
    x_ref[...] = x
    xn1v = oh * norm
    xn_ref[0, :, :] = xn1v[:, :IN2]
    xn_ref[1, :, :] = xn1v[:, IN2:]
    norm_ref[...] = norm


_prep_call = pl.pallas_call(
    _prep_body,
    out_shape=[
        jax.ShapeDtypeStruct((N, HID), jnp.float32),
        jax.ShapeDtypeStruct((NC, N, IN2), jnp.float32),
        jax.ShapeDtypeStruct((N, 1), jnp.float32),
    ],
)


def _dot_bf16(a, b):
    return jnp.dot(a.astype(jnp.bfloat16), b.astype(jnp.bfloat16),
                   preferred_element_type=jnp.float32)


def _layer1_body(x_ref, ap_ref, norm_ref, emb_ref, W_ref, b_ref, g_ref,
                 be_ref, xo_ref, xn_ref):
    norm = norm_ref[...]
    agg32 = jnp.concatenate([ap_ref[0, :N, :], ap_ref[1, :N, :]], axis=1)
    agg = norm * jnp.dot(agg32, emb_ref[...],
                         preferred_element_type=jnp.float32,
                         precision=lax.Precision.HIGHEST)
    y = _dot_bf16(agg, W_ref[...]) + b_ref[...]
    mean = jnp.mean(y, axis=0, keepdims=True)
    yc = y - mean
    var = jnp.mean(yc * yc, axis=0, keepdims=True)
    y = yc * lax.rsqrt(var + 1e-5) * g_ref[...] + be_ref[...]
    y = jnp.maximum(y, 0.0)
    xo = x_ref[...] + y
    xo_ref[...] = xo
    xnv = xo * norm
    xn_ref[0, :, :] = xnv[:, :HID2]
    xn_ref[1, :, :] = xnv[:, HID2:]


_layer1_call = pl.pallas_call(
    _layer1_body,
    out_shape=[
        jax.ShapeDtypeStruct((N, HID), jnp.float32),
        jax.ShapeDtypeStruct((NC, N, HID2), jnp.float32),
    ],
)


def _layer_body(x_ref, ap_ref, norm_ref, W_ref, b_ref, g_ref, be_ref,
                xo_ref, xn_ref):
    norm = norm_ref[...]
    agg = jnp.concatenate([ap_ref[0, :N, :], ap_ref[1, :N, :]], axis=1) * norm
    y = _dot_bf16(agg, W_ref[...]) + b_ref[...]
    mean = jnp.mean(y, axis=0, keepdims=True)
    yc = y - mean
    var = jnp.mean(yc * yc, axis=0, keepdims=True)
    y = yc * lax.rsqrt(var + 1e-5) * g_ref[...] + be_ref[...]
    y = jnp.maximum(y, 0.0)
    xo = x_ref[...] + y
    xo_ref[...] = xo
    xnv = xo * norm
    xn_ref[0, :, :] = xnv[:, :HID2]
    xn_ref[1, :, :] = xnv[:, HID2:]


_layer_call = pl.pallas_call(
    _layer_body,
    out_shape=[
        jax.ShapeDtypeStruct((N, HID), jnp.float32),
        jax.ShapeDtypeStruct((NC, N, HID2), jnp.float32),
    ],
)


def _mlp_body(x_ref, W0_ref, b0_ref, W1_ref, b1_ref, W2_ref, b2_ref, o_ref):
    z = _dot_bf16(x_ref[...], W0_ref[...])
    z = jnp.maximum(z + b0_ref[...], 0.0)
    z = _dot_bf16(z, W1_ref[...])
    z = jnp.maximum(z + b1_ref[...], 0.0)
    o_ref[...] = _dot_bf16(z, W2_ref[...]) + b2_ref[...]


_mlp_call = pl.pallas_call(
    _mlp_body,
    out_shape=jax.ShapeDtypeStruct((N, NCLS), jnp.float32),
)


def kernel(h, edge_index, e, emb, W_layers, b_layers, bn_scale, bn_bias,
           mlp_W0, mlp_b0, mlp_W1, mlp_b1, mlp_W2, mlp_b2):
    src = edge_index[0].astype(jnp.int32)
    dst = edge_index[1].astype(jnp.int32)
    pad = EP - E
    src_flat = jnp.concatenate([src, jnp.zeros((pad,), jnp.int32)])
    dummy = N + (jnp.arange(pad, dtype=jnp.int32) % (NP - N))
    dst_flat = jnp.concatenate([dst, dummy])
    src_p = src_flat.reshape(NW, NCHUNK, CHUNK)
    dst_p = dst_flat.reshape(NW, NCHUNK, CHUNK)
    src_p2 = src_flat.reshape(NS, NWIN, W, SCH)
    dst_p2 = dst_flat.reshape(NS, NWIN, W, SCH)
    zeros2 = jnp.zeros((ROWS_PER_SUB, HID2), jnp.float32)
    zeros16 = jnp.zeros((ROWS_PER_SUB, IN2), jnp.float32)

    degp = _deg_kernel(dst_p).reshape(NC, NP)
    degp_t = jnp.transpose(degp[:, :N])
    hv = h.astype(jnp.int32).reshape(N, 1)
    x, xn1, norm = _prep_call(hv, degp_t, emb)

    aggp32 = _agg16(xn1, src_p2, dst_p2, zeros16)
    x, xn = _layer1_call(x, aggp32, norm, emb, W_layers[0],
                         b_layers[0].reshape(1, HID),
                         bn_scale[0].reshape(1, HID),
                         bn_bias[0].reshape(1, HID))

    for i in range(1, NLAYERS):
        aggp = _agg64(xn, src_p2, dst_p2, zeros2)
        x, xn = _layer_call(x, aggp, norm, W_layers[i],
                            b_layers[i].reshape(1, HID),
                            bn_scale[i].reshape(1, HID),
                            bn_bias[i].reshape(1, HID))

    out = _mlp_call(x, mlp_W0, mlp_b0.reshape(1, -1),
                    mlp_W1, mlp_b1.reshape(1, -1),
                    mlp_W2, mlp_b2.reshape(1, -1))
    return out


# speedup vs baseline: 1.3976x; 1.0990x over previous
import functools

import jax
import jax.numpy as jnp
from jax import lax
from jax.experimental import pallas as pl
from jax.experimental.pallas import tpu as pltpu
from jax.experimental.pallas import tpu_sc as plsc

N = 10000
E = 320000
IN_DIM = 32
HID = 128
NLAYERS = 4
NCLS = 6

NC = 2
NS = 16
NW = NC * NS
CHUNK = 128
NCHUNK = 80
EPW = NCHUNK * CHUNK
EP = NW * EPW
HID2 = HID // 2
SCH = 512
NSUP = EP // (NS * SCH)
W = 4
NWIN = NSUP // W
NT = NWIN // 2
ROWS_PER_SUB = 640
NP = ROWS_PER_SUB * NS

_mesh = plsc.VectorSubcoreMesh(
    core_axis_name="c", subcore_axis_name="s", num_cores=NC, num_subcores=NS)


@functools.partial(
    pl.kernel,
    out_type=jax.ShapeDtypeStruct((NC * NP,), jnp.float32),
    mesh=_mesh,
    scratch_types=[
        pltpu.VMEM((NCHUNK, CHUNK), jnp.int32),
        pltpu.VMEM((CHUNK,), jnp.float32),
        pltpu.VMEM((ROWS_PER_SUB,), jnp.float32),
        pltpu.VMEM_SHARED((NP,), jnp.float32),
    ],
)
def _deg_kernel(dst_hbm, out_hbm, dst_v, ones_v, buf_v, deg_sh):
    c = lax.axis_index("c")
    s = lax.axis_index("s")
    wid = s * NC + c
    for i in range(ROWS_PER_SUB // 16):
        buf_v[pl.ds(i * 16, 16)] = jnp.zeros((16,), jnp.float32)
    pltpu.sync_copy(buf_v, deg_sh.at[pl.ds(s * ROWS_PER_SUB, ROWS_PER_SUB)])
    pltpu.sync_copy(dst_hbm.at[wid], dst_v)
    for i in range(CHUNK // 16):
        ones_v[pl.ds(i * 16, 16)] = jnp.ones((16,), jnp.float32)
    plsc.subcore_barrier()

    def body(j, carry):
        pltpu.sync_copy(ones_v, deg_sh.at[dst_v.at[j]], add=True)
        return carry

    lax.fori_loop(0, NCHUNK, body, 0)
    plsc.subcore_barrier()
    pltpu.sync_copy(deg_sh.at[pl.ds(s * ROWS_PER_SUB, ROWS_PER_SUB)], buf_v)
    pltpu.sync_copy(buf_v,
                    out_hbm.at[pl.ds(c * NP + s * ROWS_PER_SUB, ROWS_PER_SUB)])


def _make_agg(ncols):
  @functools.partial(
      pl.kernel,
      out_type=jax.ShapeDtypeStruct((NC, NP, ncols), jnp.float32),
      mesh=_mesh,
      compiler_params=pltpu.CompilerParams(use_tc_tiling_on_sc=False),
      scratch_types=[
          [pltpu.VMEM((W, SCH), jnp.int32)] * 2,
          [pltpu.VMEM((W, SCH), jnp.int32)] * 2,
          [pltpu.VMEM((SCH, ncols), jnp.float32)] * 2,
          pltpu.VMEM_SHARED((NP, ncols), jnp.float32),
          [pltpu.SemaphoreType.DMA] * 2,
          [pltpu.SemaphoreType.DMA] * 2,
          [pltpu.SemaphoreType.DMA] * 2,
      ],
  )
  def _agg_kernel(xn_hbm, src_hbm, dst_hbm, zeros_hbm, out_hbm,
                  srcw, dstw, rows, acc_sh, gsem, isrc, idst):
    c = lax.axis_index("c")
    s = lax.axis_index("s")
    pltpu.sync_copy(zeros_hbm, acc_sh.at[pl.ds(s * ROWS_PER_SUB, ROWS_PER_SUB)])
    pltpu.sync_copy(src_hbm.at[s, 0], srcw[0])
    pltpu.sync_copy(dst_hbm.at[s, 0], dstw[0])
    pltpu.async_copy(src_hbm.at[s, 1], srcw[1], isrc[1])
    pltpu.async_copy(dst_hbm.at[s, 1], dstw[1], idst[1])
    plsc.subcore_barrier()

    xnc = xn_hbm.at[c]
    pltpu.async_copy(xnc.at[srcw[0].at[0]], rows[0], gsem[0])
    pltpu.async_copy(xnc.at[srcw[0].at[1]], rows[1], gsem[1])

    def body(t, carry):
        for p in range(2):
            if p == 0:
                @pl.when(t > 0)
                def _():
                    pltpu.make_async_copy(dst_hbm.at[s, 0], dstw[0], idst[0]).wait()
            else:
                pltpu.make_async_copy(dst_hbm.at[s, 0], dstw[1], idst[1]).wait()
            for k in range(W):
                b = k % 2
                pltpu.make_async_copy(xnc.at[srcw[p].at[k]], rows[b], gsem[b]).wait()
                pltpu.sync_copy(rows[b], acc_sh.at[dstw[p].at[k]], add=True)
                if k < 2:
                    pltpu.async_copy(xnc.at[srcw[p].at[k + 2]], rows[b], gsem[b])
                elif p == 0:
                    if k == 2:
                        pltpu.make_async_copy(src_hbm.at[s, 1], srcw[1], isrc[1]).wait()
                    pltpu.async_copy(xnc.at[srcw[1].at[k - 2]], rows[b], gsem[b])
                else:
                    @pl.when(t < NT - 1)
                    def _(b=b, k=k):
                        if k == 2:
                            pltpu.make_async_copy(
                                src_hbm.at[s, 0], srcw[0], isrc[0]).wait()
                        pltpu.async_copy(xnc.at[srcw[0].at[k - 2]], rows[b], gsem[b])
            @pl.when(t < NT - 1)
            def _(p=p, t=t):
                wn = 2 * t + 2 + p
                pltpu.async_copy(src_hbm.at[s, wn], srcw[p], isrc[p])
                pltpu.async_copy(dst_hbm.at[s, wn], dstw[p], idst[p])
        return carry

    lax.fori_loop(0, NT, body, 0)
    plsc.subcore_barrier()
    pltpu.sync_copy(acc_sh.at[pl.ds(s * ROWS_PER_SUB, ROWS_PER_SUB)],
                    out_hbm.at[c, pl.ds(s * ROWS_PER_SUB, ROWS_PER_SUB)])

  return _agg_kernel


IN2 = IN_DIM // NC
_agg64 = _make_agg(HID2)
_agg16 = _make_agg(IN2)


def _prep_body(h_ref, degp_ref, emb_ref, x_ref, xn_ref, norm_ref):
    d = degp_ref[...]
    deg = d[:, 0:1] + d[:, 1:2]
    norm = lax.rsqrt(jnp.maximum(deg, 1.0))
    hv = h_ref[...]
    oh = (hv == lax.broadcasted_iota(jnp.int32, (N, IN_DIM), 1)).astype(jnp.float32)
    x = jnp.dot(oh, emb_ref[...], preferred_element_type=jnp.float32, precision=lax.Precision.HIGHEST)
    x_ref[...] = x
    xn1v = oh * norm
    xn_ref[0, :, :] = xn1v[:, :IN2]
    xn_ref[1, :, :] = xn1v[:, IN2:]
    norm_ref[...] = norm


_prep_call = pl.pallas_call(
    _prep_body,
    out_shape=[
        jax.ShapeDtypeStruct((N, HID), jnp.float32),
        jax.ShapeDtypeStruct((NC, N, IN2), jnp.float32),
        jax.ShapeDtypeStruct((N, 1), jnp.float32),
    ],
)


def _dot_bf16(a, b):
    return jnp.dot(a.astype(jnp.bfloat16), b.astype(jnp.bfloat16),
                   preferred_element_type=jnp.float32)


def _layer1_body(x_ref, ap_ref, norm_ref, emb_ref, W_ref, b_ref, g_ref,
                 be_ref, xo_ref, xn_ref):
    norm = norm_ref[...]
    agg32 = jnp.concatenate([ap_ref[0, :N, :], ap_ref[1, :N, :]], axis=1)
    agg = norm * jnp.dot(agg32, emb_ref[...],
                         preferred_element_type=jnp.float32,
                         precision=lax.Precision.HIGHEST)
    y = _dot_bf16(agg, W_ref[...]) + b_ref[...]
    mean = jnp.mean(y, axis=0, keepdims=True)
    yc = y - mean
    var = jnp.mean(yc * yc, axis=0, keepdims=True)
    y = yc * lax.rsqrt(var + 1e-5) * g_ref[...] + be_ref[...]
    y = jnp.maximum(y, 0.0)
    xo = x_ref[...] + y
    xo_ref[...] = xo
    xnv = xo * norm
    xn_ref[0, :, :] = xnv[:, :HID2]
    xn_ref[1, :, :] = xnv[:, HID2:]


_layer1_call = pl.pallas_call(
    _layer1_body,
    out_shape=[
        jax.ShapeDtypeStruct((N, HID), jnp.float32),
        jax.ShapeDtypeStruct((NC, N, HID2), jnp.float32),
    ],
)


def _layer_body(x_ref, ap_ref, norm_ref, W_ref, b_ref, g_ref, be_ref,
                xo_ref, xn_ref):
    norm = norm_ref[...]
    agg = jnp.concatenate([ap_ref[0, :N, :], ap_ref[1, :N, :]], axis=1) * norm
    y = _dot_bf16(agg, W_ref[...]) + b_ref[...]
    mean = jnp.mean(y, axis=0, keepdims=True)
    yc = y - mean
    var = jnp.mean(yc * yc, axis=0, keepdims=True)
    y = yc * lax.rsqrt(var + 1e-5) * g_ref[...] + be_ref[...]
    y = jnp.maximum(y, 0.0)
    xo = x_ref[...] + y
    xo_ref[...] = xo
    xnv = xo * norm
    xn_ref[0, :, :] = xnv[:, :HID2]
    xn_ref[1, :, :] = xnv[:, HID2:]


_layer_call = pl.pallas_call(
    _layer_body,
    out_shape=[
        jax.ShapeDtypeStruct((N, HID), jnp.float32),
        jax.ShapeDtypeStruct((NC, N, HID2), jnp.float32),
    ],
)


def _layer_mlp_body(x_ref, ap_ref, norm_ref, W_ref, b_ref, g_ref, be_ref,
                    W0_ref, b0_ref, W1_ref, b1_ref, W2_ref, b2_ref, o_ref):
    norm = norm_ref[...]
    agg = jnp.concatenate([ap_ref[0, :N, :], ap_ref[1, :N, :]], axis=1) * norm
    y = _dot_bf16(agg, W_ref[...]) + b_ref[...]
    mean = jnp.mean(y, axis=0, keepdims=True)
    yc = y - mean
    var = jnp.mean(yc * yc, axis=0, keepdims=True)
    y = yc * lax.rsqrt(var + 1e-5) * g_ref[...] + be_ref[...]
    y = jnp.maximum(y, 0.0)
    xo = x_ref[...] + y
    z = _dot_bf16(xo, W0_ref[...])
    z = jnp.maximum(z + b0_ref[...], 0.0)
    z = _dot_bf16(z, W1_ref[...])
    z = jnp.maximum(z + b1_ref[...], 0.0)
    o_ref[...] = _dot_bf16(z, W2_ref[...]) + b2_ref[...]


_layer_mlp_call = pl.pallas_call(
    _layer_mlp_body,
    out_shape=jax.ShapeDtypeStruct((N, NCLS), jnp.float32),
)


def kernel(h, edge_index, e, emb, W_layers, b_layers, bn_scale, bn_bias,
           mlp_W0, mlp_b0, mlp_W1, mlp_b1, mlp_W2, mlp_b2):
    src = edge_index[0].astype(jnp.int32)
    dst = edge_index[1].astype(jnp.int32)
    pad = EP - E
    src_flat = jnp.concatenate([src, jnp.zeros((pad,), jnp.int32)])
    dummy = N + (jnp.arange(pad, dtype=jnp.int32) % (NP - N))
    dst_flat = jnp.concatenate([dst, dummy])
    src_p = src_flat.reshape(NW, NCHUNK, CHUNK)
    dst_p = dst_flat.reshape(NW, NCHUNK, CHUNK)
    src_p2 = src_flat.reshape(NS, NWIN, W, SCH)
    dst_p2 = dst_flat.reshape(NS, NWIN, W, SCH)
    zeros2 = jnp.zeros((ROWS_PER_SUB, HID2), jnp.float32)
    zeros16 = jnp.zeros((ROWS_PER_SUB, IN2), jnp.float32)

    degp = _deg_kernel(dst_p).reshape(NC, NP)
    degp_t = jnp.transpose(degp[:, :N])
    hv = h.astype(jnp.int32).reshape(N, 1)
    x, xn1, norm = _prep_call(hv, degp_t, emb)

    aggp32 = _agg16(xn1, src_p2, dst_p2, zeros16)
    x, xn = _layer1_call(x, aggp32, norm, emb, W_layers[0],
                         b_layers[0].reshape(1, HID),
                         bn_scale[0].reshape(1, HID),
                         bn_bias[0].reshape(1, HID))

    for i in range(1, NLAYERS - 1):
        aggp = _agg64(xn, src_p2, dst_p2, zeros2)
        x, xn = _layer_call(x, aggp, norm, W_layers[i],
                            b_layers[i].reshape(1, HID),
                            bn_scale[i].reshape(1, HID),
                            bn_bias[i].reshape(1, HID))

    i = NLAYERS - 1
    aggp = _agg64(xn, src_p2, dst_p2, zeros2)
    out = _layer_mlp_call(x, aggp, norm, W_layers[i],
                          b_layers[i].reshape(1, HID),
                          bn_scale[i].reshape(1, HID),
                          bn_bias[i].reshape(1, HID),
                          mlp_W0, mlp_b0.reshape(1, -1),
                          mlp_W1, mlp_b1.reshape(1, -1),
                          mlp_W2, mlp_b2.reshape(1, -1))
    return out
